# baseline (speedup 1.0000x reference)
import functools
import math

import jax
import jax.numpy as jnp
from jax import lax
from jax.experimental import pallas as pl
from jax.experimental.pallas import tpu as pltpu
from jax.experimental.pallas import tpu_sc as plsc

_N = 100000
_B = 1024
_SMOOTHING = 0.1
_CONF = 1.0 - _SMOOTHING
_FILL = _SMOOTHING / (_N - 2)
_C1 = _FILL * (_N - 2) * math.log(_FILL) + _CONF * math.log(_CONF)


_NW = 32
_BPW = _B // _NW


def _sc_fetch_segments(x2d, t32):
    mesh = plsc.VectorSubcoreMesh(core_axis_name="c", subcore_axis_name="s")

    @functools.partial(
        pl.kernel,
        mesh=mesh,
        out_type=jax.ShapeDtypeStruct((_B, 128), jnp.float32),
        scratch_types=[
            pltpu.VMEM((_BPW,), jnp.int32),
            pltpu.VMEM((_BPW, 8, 128), jnp.float32),
            pltpu.VMEM((_BPW, 128), jnp.float32),
            pltpu.SemaphoreType.DMA,
        ],
    )
    def k(x_hbm, t_hbm, out_hbm, t_v, seg_v, stage_v, sem):
        wid = lax.axis_index("s") * 2 + lax.axis_index("c")
        base = wid * _BPW
        pltpu.sync_copy(t_hbm.at[pl.ds(base, _BPW)], t_v)
        tvecs = [t_v[pl.ds(16 * c, 16)] for c in range(_BPW // 16)]
        copies = []
        for j in range(_BPW):
            tj = tvecs[j // 16][j % 16]
            aj = pl.multiple_of((tj >> 7) << 7, 128)
            rj = pl.multiple_of(base + (j & ~7), 8)
            copies.append(pltpu.async_copy(
                x_hbm.at[pl.ds(rj, 8), pl.ds(aj, 128)], seg_v.at[j], sem))
        for c in copies:
            c.wait()
        for j in range(_BPW):
            for q in range(8):
                stage_v[j, pl.ds(16 * q, 16)] = (
                    seg_v[j, j % 8, pl.ds(16 * q, 16)])
        pltpu.sync_copy(stage_v, out_hbm.at[pl.ds(base, _BPW), :])

    return k(x2d, t32)


_BR = 64
_NBLK = _B // _BR


def _tc_body(x_ref, seg_ref, tm_ref, v_ref, out_ref, acc_ref):
    i = pl.program_id(0)
    x = x_ref[...]
    v = v_ref[...]
    rs = jnp.sum(x, axis=1, keepdims=True) - x[:, 0:1]
    lane = lax.broadcasted_iota(jnp.int32, (_BR, 128), 1)
    gmask = (lane == tm_ref[...]) & (v > 0.0)
    g_part = jnp.sum(jnp.where(gmask, seg_ref[...], 0.0))
    part = (_C1 * jnp.sum(v) - _FILL * jnp.sum(rs * v)
            + (_FILL - _CONF) * g_part)

    @pl.when(i == 0)
    def _():
        acc_ref[0, 0] = part

    @pl.when(i > 0)
    def _():
        acc_ref[0, 0] += part

    @pl.when(i == _NBLK - 1)
    def _():
        out_ref[0, 0] = acc_ref[0, 0] * (1.0 / _B)


def _tc_reduce(x, seg, tmod, vcol, interpret=False):
    return pl.pallas_call(
        _tc_body,
        grid=(_NBLK,),
        in_specs=[
            pl.BlockSpec((_BR, _N), lambda i: (i, 0)),
            pl.BlockSpec((_BR, 128), lambda i: (i, 0)),
            pl.BlockSpec((_BR, 1), lambda i: (i, 0)),
            pl.BlockSpec((_BR, 1), lambda i: (i, 0)),
        ],
        out_specs=pl.BlockSpec(memory_space=pltpu.SMEM),
        out_shape=jax.ShapeDtypeStruct((1, 1), jnp.float32),
        scratch_shapes=[pltpu.SMEM((1, 1), jnp.float32)],
        compiler_params=pltpu.CompilerParams(
            dimension_semantics=("arbitrary",)),
        interpret=interpret,
    )(x, seg, tmod, vcol)


def kernel(log_inputs, targets):
    t32 = targets.astype(jnp.int32)
    seg = _sc_fetch_segments(log_inputs, t32)
    tmod = (t32 & 127).reshape(_B, 1)
    vcol = (t32 != 0).astype(jnp.float32).reshape(_B, 1)
    out = _tc_reduce(log_inputs, seg, tmod, vcol)
    return out.reshape(())


# speedup vs baseline: 1.0030x; 1.0030x over previous
import functools
import math

import jax
import jax.numpy as jnp
from jax import lax
from jax.experimental import pallas as pl
from jax.experimental.pallas import tpu as pltpu
from jax.experimental.pallas import tpu_sc as plsc

_N = 100000
_B = 1024
_SMOOTHING = 0.1
_CONF = 1.0 - _SMOOTHING
_FILL = _SMOOTHING / (_N - 2)
_C1 = _FILL * (_N - 2) * math.log(_FILL) + _CONF * math.log(_CONF)


_NW = 32
_BPW = _B // _NW


def _sc_fetch_segments(x2d, t32):
    mesh = plsc.VectorSubcoreMesh(core_axis_name="c", subcore_axis_name="s")

    @functools.partial(
        pl.kernel,
        mesh=mesh,
        out_type=jax.ShapeDtypeStruct((_B, 128), jnp.float32),
        scratch_types=[
            pltpu.VMEM((_BPW,), jnp.int32),
            pltpu.VMEM((_BPW, 8, 128), jnp.float32),
            pltpu.VMEM((_BPW, 128), jnp.float32),
            pltpu.SemaphoreType.DMA,
        ],
    )
    def k(x_hbm, t_hbm, out_hbm, t_v, seg_v, stage_v, sem):
        wid = lax.axis_index("s") * 2 + lax.axis_index("c")
        base = wid * _BPW
        pltpu.sync_copy(t_hbm.at[pl.ds(base, _BPW)], t_v)
        tvecs = [t_v[pl.ds(16 * c, 16)] for c in range(_BPW // 16)]
        copies = []
        for j in range(_BPW):
            tj = tvecs[j // 16][j % 16]
            aj = pl.multiple_of((tj >> 7) << 7, 128)
            rj = pl.multiple_of(base + (j & ~7), 8)
            copies.append(pltpu.async_copy(
                x_hbm.at[pl.ds(rj, 8), pl.ds(aj, 128)], seg_v.at[j], sem))
        for c in copies:
            c.wait()
        for j in range(_BPW):
            for q in range(8):
                stage_v[j, pl.ds(16 * q, 16)] = (
                    seg_v[j, j % 8, pl.ds(16 * q, 16)])
        pltpu.sync_copy(stage_v, out_hbm.at[pl.ds(base, _BPW), :])

    return k(x2d, t32)


_BR = 16
_NS = 4
_NBLK = _B // (_BR * _NS)


def _tc_body(*refs):
    x_refs = refs[:_NS]
    seg_ref, tm_ref, v_ref, out_ref, acc_ref = refs[_NS:]
    i = pl.program_id(0)
    v = v_ref[...]
    lane = lax.broadcasted_iota(jnp.int32, (_NS * _BR, 128), 1)
    gmask = (lane == tm_ref[...]) & (v > 0.0)
    g_part = jnp.sum(jnp.where(gmask, seg_ref[...], 0.0))
    part = _C1 * jnp.sum(v) + (_FILL - _CONF) * g_part
    for k, x_ref in enumerate(x_refs):
        x = x_ref[...]
        rs = jnp.sum(x, axis=1, keepdims=True) - x[:, 0:1]
        part = part - _FILL * jnp.sum(rs * v[k * _BR:(k + 1) * _BR, :])

    @pl.when(i == 0)
    def _():
        acc_ref[0, 0] = part

    @pl.when(i > 0)
    def _():
        acc_ref[0, 0] += part

    @pl.when(i == _NBLK - 1)
    def _():
        out_ref[0, 0] = acc_ref[0, 0] * (1.0 / _B)


def _make_x_spec(k):
    return pl.BlockSpec((_BR, _N), lambda i: (_NS * i + k, 0))


def _tc_reduce(x, seg, tmod, vcol, interpret=False):
    return pl.pallas_call(
        _tc_body,
        grid=(_NBLK,),
        in_specs=[_make_x_spec(k) for k in range(_NS)] + [
            pl.BlockSpec((_NS * _BR, 128), lambda i: (i, 0)),
            pl.BlockSpec((_NS * _BR, 1), lambda i: (i, 0)),
            pl.BlockSpec((_NS * _BR, 1), lambda i: (i, 0)),
        ],
        out_specs=pl.BlockSpec(memory_space=pltpu.SMEM),
        out_shape=jax.ShapeDtypeStruct((1, 1), jnp.float32),
        scratch_shapes=[pltpu.SMEM((1, 1), jnp.float32)],
        compiler_params=pltpu.CompilerParams(
            dimension_semantics=("arbitrary",)),
        interpret=interpret,
    )(*([x] * _NS), seg, tmod, vcol)


def kernel(log_inputs, targets):
    t32 = targets.astype(jnp.int32)
    seg = _sc_fetch_segments(log_inputs, t32)
    tmod = (t32 & 127).reshape(_B, 1)
    vcol = (t32 != 0).astype(jnp.float32).reshape(_B, 1)
    out = _tc_reduce(log_inputs, seg, tmod, vcol)
    return out.reshape(())
